# Initial kernel scaffold; baseline (speedup 1.0000x reference)
import jax
import jax.numpy as jnp
from jax.experimental import pallas as pl


def kernel(x, tgt_uid_table, tgt_iid_table, rp_W, lin_W):
    raise NotImplementedError("write your pallas kernel here")


# speedup vs baseline: 31.8255x; 31.8255x over previous
import functools

import jax
import jax.numpy as jnp
from jax import lax
from jax.experimental import pallas as pl
from jax.experimental.pallas import tpu as pltpu

_TARGET = 5.0
_K = 10000
_D = 128
_N = 100000
_B = 64
_RB = 2048
_NBLK = 49
_NPAD = _RB * _NBLK
_MAXBITS = 0x7FFFFFFF


def _gather_w_kernel(iid_ref, tbl_ref, lin_ref, w_ref):
    w_ref[...] = tbl_ref[...] * lin_ref[...]


def _gather_w(iid, tgt_iid_table, lin_W):
    grid_spec = pltpu.PrefetchScalarGridSpec(
        num_scalar_prefetch=1,
        grid=(_B,),
        in_specs=[
            pl.BlockSpec((1, 1, _D), lambda i, iid_p: (iid_p[i], 0, 0)),
            pl.BlockSpec((1, _D), lambda i, iid_p: (0, 0)),
        ],
        out_specs=pl.BlockSpec((1, 1, _D), lambda i, iid_p: (i, 0, 0)),
    )
    out = pl.pallas_call(
        _gather_w_kernel,
        grid_spec=grid_spec,
        out_shape=jax.ShapeDtypeStruct((_B, 1, _D), jnp.float32),
    )(iid, tgt_iid_table.reshape(-1, 1, _D), lin_W)
    return out.reshape(_B, _D)


def _score_kernel(feat_ref, rpw_ref, w_ref, uid_ref, bits_ref, vt_ref):
    i = pl.program_id(0)
    srp = lax.dot_general(
        feat_ref[...], rpw_ref[...], (((1,), (1,)), ((), ())),
        preferred_element_type=jnp.float32,
        precision=lax.Precision.HIGHEST,
    )
    a = lax.dot_general(
        srp, uid_ref[...], (((1,), (1,)), ((), ())),
        preferred_element_type=jnp.float32,
        precision=lax.Precision.HIGHEST,
    )
    v = lax.dot_general(
        w_ref[...], uid_ref[...], (((1,), (1,)), ((), ())),
        preferred_element_type=jnp.float32,
        precision=lax.Precision.HIGHEST,
    )
    bits = lax.bitcast_convert_type(jnp.abs(a - _TARGET), jnp.int32)
    col = i * _RB + lax.broadcasted_iota(jnp.int32, (_B, _RB), 1)
    valid = col < _N
    bits_ref[...] = jnp.where(valid, bits, _MAXBITS)
    vt_ref[...] = jnp.where(valid, v, 0.0)


def _score(feat, rp_W, w, tgt_uid_table):
    return pl.pallas_call(
        _score_kernel,
        grid=(_NBLK,),
        in_specs=[
            pl.BlockSpec((_B, _D), lambda i: (0, 0)),
            pl.BlockSpec((_D, _D), lambda i: (0, 0)),
            pl.BlockSpec((_B, _D), lambda i: (0, 0)),
            pl.BlockSpec((_RB, _D), lambda i: (i, 0)),
        ],
        out_specs=[
            pl.BlockSpec((_B, _RB), lambda i: (0, i)),
            pl.BlockSpec((_B, _RB), lambda i: (0, i)),
        ],
        out_shape=[
            jax.ShapeDtypeStruct((_B, _NPAD), jnp.int32),
            jax.ShapeDtypeStruct((_B, _NPAD), jnp.float32),
        ],
    )(feat, rp_W, w, tgt_uid_table)


def _select_kernel(bits_ref, vt_ref, out_ref):
    bits = bits_ref[...]

    def body(j, p):
        test = p | jnp.left_shift(jnp.int32(1), 30 - j)
        cnt = jnp.sum((bits < test).astype(jnp.int32), axis=1, keepdims=True)
        return jnp.where(cnt < _K, test, p)

    t = lax.fori_loop(0, 31, body, jnp.zeros((bits.shape[0], 1), jnp.int32))

    v = vt_ref[...]
    lt = bits < t
    eq = bits == t
    cnt_lt = jnp.sum(lt.astype(jnp.int32), axis=1)
    cnt_eq = jnp.sum(eq.astype(jnp.int32), axis=1)
    sum_lt = jnp.sum(jnp.where(lt, v, 0.0), axis=1)
    sum_eq = jnp.sum(jnp.where(eq, v, 0.0), axis=1)
    needed = (_K - cnt_lt).astype(jnp.float32)
    res = (sum_lt + needed * sum_eq / cnt_eq.astype(jnp.float32)) / _K
    out_ref[...] = jnp.broadcast_to(res[:, None], out_ref.shape)


def _select(bits, vt):
    nprog = 4
    cb = _B // nprog
    out = pl.pallas_call(
        _select_kernel,
        grid=(nprog,),
        in_specs=[
            pl.BlockSpec((cb, _NPAD), lambda i: (i, 0)),
            pl.BlockSpec((cb, _NPAD), lambda i: (i, 0)),
        ],
        out_specs=pl.BlockSpec((cb, 128), lambda i: (i, 0)),
        out_shape=jax.ShapeDtypeStruct((_B, 128), jnp.float32),
    )(bits, vt)
    return out[:, 0]


@jax.jit
def kernel(x, tgt_uid_table, tgt_iid_table, rp_W, lin_W):
    iid = x[:, 0].astype(jnp.int32)
    feat = x[:, 1:_D + 1]
    w = _gather_w(iid, tgt_iid_table, lin_W)
    bits, vt = _score(feat, rp_W, w, tgt_uid_table)
    return _select(bits, vt)


# speedup vs baseline: 37.5861x; 1.1810x over previous
import functools

import jax
import jax.numpy as jnp
from jax import lax
from jax.experimental import pallas as pl
from jax.experimental.pallas import tpu as pltpu

_TARGET = 5.0
_K = 10000
_D = 128
_N = 100000
_B = 64
_RB = 2048
_NBLK = 49
_NPAD = _RB * _NBLK
_MAXBITS = 0x7FFFFFFF


def _gather_w_kernel(iid_ref, tbl_ref, lin_ref, w_ref):
    w_ref[...] = tbl_ref[...] * lin_ref[...]


def _gather_w(iid, tgt_iid_table, lin_W):
    grid_spec = pltpu.PrefetchScalarGridSpec(
        num_scalar_prefetch=1,
        grid=(_B,),
        in_specs=[
            pl.BlockSpec((1, 1, _D), lambda i, iid_p: (iid_p[i], 0, 0)),
            pl.BlockSpec((1, _D), lambda i, iid_p: (0, 0)),
        ],
        out_specs=pl.BlockSpec((1, 1, _D), lambda i, iid_p: (i, 0, 0)),
    )
    out = pl.pallas_call(
        _gather_w_kernel,
        grid_spec=grid_spec,
        out_shape=jax.ShapeDtypeStruct((_B, 1, _D), jnp.float32),
    )(iid, tgt_iid_table.reshape(-1, 1, _D), lin_W)
    return out.reshape(_B, _D)


def _score_kernel(feat_ref, rpw_ref, w_ref, uid_ref, bits_ref, vt_ref):
    i = pl.program_id(0)
    srp = lax.dot_general(
        feat_ref[...], rpw_ref[...], (((1,), (1,)), ((), ())),
        preferred_element_type=jnp.float32,
        precision=lax.Precision.DEFAULT,
    )
    a = lax.dot_general(
        srp, uid_ref[...], (((1,), (1,)), ((), ())),
        preferred_element_type=jnp.float32,
        precision=lax.Precision.DEFAULT,
    )
    v = lax.dot_general(
        w_ref[...], uid_ref[...], (((1,), (1,)), ((), ())),
        preferred_element_type=jnp.float32,
        precision=lax.Precision.HIGHEST,
    )
    bits = lax.bitcast_convert_type(jnp.abs(a - _TARGET), jnp.int32)
    col = i * _RB + lax.broadcasted_iota(jnp.int32, (_B, _RB), 1)
    valid = col < _N
    bits_ref[...] = jnp.where(valid, bits, _MAXBITS)
    vt_ref[...] = jnp.where(valid, v, 0.0)


def _score(feat, rp_W, w, tgt_uid_table):
    return pl.pallas_call(
        _score_kernel,
        grid=(_NBLK,),
        in_specs=[
            pl.BlockSpec((_B, _D), lambda i: (0, 0)),
            pl.BlockSpec((_D, _D), lambda i: (0, 0)),
            pl.BlockSpec((_B, _D), lambda i: (0, 0)),
            pl.BlockSpec((_RB, _D), lambda i: (i, 0)),
        ],
        out_specs=[
            pl.BlockSpec((_B, _RB), lambda i: (0, i)),
            pl.BlockSpec((_B, _RB), lambda i: (0, i)),
        ],
        out_shape=[
            jax.ShapeDtypeStruct((_B, _NPAD), jnp.int32),
            jax.ShapeDtypeStruct((_B, _NPAD), jnp.float32),
        ],
    )(feat, rp_W, w, tgt_uid_table)


def _select_kernel(bits_ref, vt_ref, out_ref):
    bits = bits_ref[...]

    def body(j, p):
        test = p | jnp.left_shift(jnp.int32(1), 30 - j)
        cnt = jnp.sum((bits < test).astype(jnp.int32), axis=1, keepdims=True)
        return jnp.where(cnt < _K, test, p)

    t = lax.fori_loop(0, 31, body, jnp.zeros((bits.shape[0], 1), jnp.int32))

    v = vt_ref[...]
    lt = bits < t
    eq = bits == t
    cnt_lt = jnp.sum(lt.astype(jnp.int32), axis=1)
    cnt_eq = jnp.sum(eq.astype(jnp.int32), axis=1)
    sum_lt = jnp.sum(jnp.where(lt, v, 0.0), axis=1)
    sum_eq = jnp.sum(jnp.where(eq, v, 0.0), axis=1)
    needed = (_K - cnt_lt).astype(jnp.float32)
    res = (sum_lt + needed * sum_eq / cnt_eq.astype(jnp.float32)) / _K
    out_ref[...] = jnp.broadcast_to(res[:, None], out_ref.shape)


def _select(bits, vt):
    nprog = 4
    cb = _B // nprog
    out = pl.pallas_call(
        _select_kernel,
        grid=(nprog,),
        in_specs=[
            pl.BlockSpec((cb, _NPAD), lambda i: (i, 0)),
            pl.BlockSpec((cb, _NPAD), lambda i: (i, 0)),
        ],
        out_specs=pl.BlockSpec((cb, 128), lambda i: (i, 0)),
        out_shape=jax.ShapeDtypeStruct((_B, 128), jnp.float32),
    )(bits, vt)
    return out[:, 0]


@jax.jit
def kernel(x, tgt_uid_table, tgt_iid_table, rp_W, lin_W):
    iid = x[:, 0].astype(jnp.int32)
    feat = x[:, 1:_D + 1]
    w = _gather_w(iid, tgt_iid_table, lin_W)
    bits, vt = _score(feat, rp_W, w, tgt_uid_table)
    return _select(bits, vt)


# speedup vs baseline: 38.7718x; 1.0315x over previous
import functools

import jax
import jax.numpy as jnp
from jax import lax
from jax.experimental import pallas as pl
from jax.experimental.pallas import tpu as pltpu
from jax.experimental.pallas import tpu_sc as plsc

_TARGET = 5.0
_K = 10000
_D = 128
_N = 100000
_B = 64
_RB = 2048
_NBLK = 49
_NPAD = _RB * _NBLK
_MAXBITS = 0x7FFFFFFF


_GW = 8
_RPW = _B // _GW


def _gather_w(iid, tgt_iid_table, lin_W):
    mesh = plsc.VectorSubcoreMesh(core_axis_name="c", subcore_axis_name="s")
    nc = plsc.get_sparse_core_info().num_cores

    @functools.partial(
        pl.kernel,
        mesh=mesh,
        out_type=jax.ShapeDtypeStruct((_B, _D), jnp.float32),
        scratch_types=[
            pltpu.VMEM((_RPW,), jnp.int32),
            pltpu.VMEM((_RPW, _D), jnp.float32),
            pltpu.VMEM((_D,), jnp.float32),
            pltpu.SemaphoreType.DMA,
        ],
    )
    def gw(iid_hbm, tbl_hbm, lin_hbm, w_hbm, idx_v, rows_v, lin_v, sem):
        wid = lax.axis_index("s") * nc + lax.axis_index("c")

        @pl.when(wid < _GW)
        def _():
            base = wid * _RPW
            pltpu.sync_copy(iid_hbm.at[pl.ds(base, _RPW)], idx_v)
            pltpu.sync_copy(lin_hbm, lin_v)
            pltpu.async_copy(tbl_hbm.at[idx_v], rows_v, sem).wait()
            for r in range(_RPW):
                for j in range(_D // 16):
                    s = pl.ds(j * 16, 16)
                    rows_v[r, s] = rows_v[r, s] * lin_v[s]
            pltpu.sync_copy(rows_v, w_hbm.at[pl.ds(base, _RPW)])

    return gw(iid, tgt_iid_table, lin_W.reshape(_D))


def _score_kernel(feat_ref, rpw_ref, w_ref, uid_ref, bits_ref, vt_ref):
    i = pl.program_id(0)
    srp = lax.dot_general(
        feat_ref[...], rpw_ref[...], (((1,), (1,)), ((), ())),
        preferred_element_type=jnp.float32,
        precision=lax.Precision.DEFAULT,
    )
    a = lax.dot_general(
        srp, uid_ref[...], (((1,), (1,)), ((), ())),
        preferred_element_type=jnp.float32,
        precision=lax.Precision.DEFAULT,
    )
    v = lax.dot_general(
        w_ref[...], uid_ref[...], (((1,), (1,)), ((), ())),
        preferred_element_type=jnp.float32,
        precision=lax.Precision.HIGHEST,
    )
    bits = lax.bitcast_convert_type(jnp.abs(a - _TARGET), jnp.int32)
    col = i * _RB + lax.broadcasted_iota(jnp.int32, (_B, _RB), 1)
    valid = col < _N
    bits_ref[...] = jnp.where(valid, bits, _MAXBITS)
    vt_ref[...] = jnp.where(valid, v, 0.0)


def _score(feat, rp_W, w, tgt_uid_table):
    return pl.pallas_call(
        _score_kernel,
        grid=(_NBLK,),
        in_specs=[
            pl.BlockSpec((_B, _D), lambda i: (0, 0)),
            pl.BlockSpec((_D, _D), lambda i: (0, 0)),
            pl.BlockSpec((_B, _D), lambda i: (0, 0)),
            pl.BlockSpec((_RB, _D), lambda i: (i, 0)),
        ],
        out_specs=[
            pl.BlockSpec((_B, _RB), lambda i: (0, i)),
            pl.BlockSpec((_B, _RB), lambda i: (0, i)),
        ],
        out_shape=[
            jax.ShapeDtypeStruct((_B, _NPAD), jnp.int32),
            jax.ShapeDtypeStruct((_B, _NPAD), jnp.float32),
        ],
    )(feat, rp_W, w, tgt_uid_table)


def _select_kernel(bits_ref, vt_ref, out_ref):
    bits = bits_ref[...]

    def body(j, p):
        test = p | jnp.left_shift(jnp.int32(1), 30 - j)
        cnt = jnp.sum((bits < test).astype(jnp.int32), axis=1, keepdims=True)
        return jnp.where(cnt < _K, test, p)

    t = lax.fori_loop(0, 31, body, jnp.zeros((bits.shape[0], 1), jnp.int32))

    v = vt_ref[...]
    lt = bits < t
    eq = bits == t
    cnt_lt = jnp.sum(lt.astype(jnp.int32), axis=1)
    cnt_eq = jnp.sum(eq.astype(jnp.int32), axis=1)
    sum_lt = jnp.sum(jnp.where(lt, v, 0.0), axis=1)
    sum_eq = jnp.sum(jnp.where(eq, v, 0.0), axis=1)
    needed = (_K - cnt_lt).astype(jnp.float32)
    res = (sum_lt + needed * sum_eq / cnt_eq.astype(jnp.float32)) / _K
    out_ref[...] = jnp.broadcast_to(res[:, None], out_ref.shape)


def _select(bits, vt):
    nprog = 4
    cb = _B // nprog
    out = pl.pallas_call(
        _select_kernel,
        grid=(nprog,),
        in_specs=[
            pl.BlockSpec((cb, _NPAD), lambda i: (i, 0)),
            pl.BlockSpec((cb, _NPAD), lambda i: (i, 0)),
        ],
        out_specs=pl.BlockSpec((cb, 128), lambda i: (i, 0)),
        out_shape=jax.ShapeDtypeStruct((_B, 128), jnp.float32),
    )(bits, vt)
    return out[:, 0]


@jax.jit
def kernel(x, tgt_uid_table, tgt_iid_table, rp_W, lin_W):
    iid = x[:, 0].astype(jnp.int32)
    feat = x[:, 1:_D + 1]
    w = _gather_w(iid, tgt_iid_table, lin_W)
    bits, vt = _score(feat, rp_W, w, tgt_uid_table)
    return _select(bits, vt)


# speedup vs baseline: 43.1562x; 1.1131x over previous
import functools

import jax
import jax.numpy as jnp
from jax import lax
from jax.experimental import pallas as pl
from jax.experimental.pallas import tpu as pltpu
from jax.experimental.pallas import tpu_sc as plsc

_TARGET = 5.0
_K = 10000
_D = 128
_N = 100000
_B = 64
_RB = 2048
_NBLK = 49
_NPAD = _RB * _NBLK
_MAXBITS = 0x7FFFFFFF


_GW = 8
_RPW = _B // _GW


def _gather_w(iid, tgt_iid_table, lin_W):
    mesh = plsc.VectorSubcoreMesh(core_axis_name="c", subcore_axis_name="s")
    nc = plsc.get_sparse_core_info().num_cores

    @functools.partial(
        pl.kernel,
        mesh=mesh,
        out_type=jax.ShapeDtypeStruct((_B, _D), jnp.float32),
        scratch_types=[
            pltpu.VMEM((_RPW,), jnp.int32),
            pltpu.VMEM((_RPW, _D), jnp.float32),
            pltpu.VMEM((_D,), jnp.float32),
            pltpu.SemaphoreType.DMA,
        ],
    )
    def gw(iid_hbm, tbl_hbm, lin_hbm, w_hbm, idx_v, rows_v, lin_v, sem):
        wid = lax.axis_index("s") * nc + lax.axis_index("c")

        @pl.when(wid < _GW)
        def _():
            base = wid * _RPW
            pltpu.sync_copy(iid_hbm.at[pl.ds(base, _RPW)], idx_v)
            pltpu.sync_copy(lin_hbm, lin_v)
            pltpu.async_copy(tbl_hbm.at[idx_v], rows_v, sem).wait()
            for r in range(_RPW):
                for j in range(_D // 16):
                    s = pl.ds(j * 16, 16)
                    rows_v[r, s] = rows_v[r, s] * lin_v[s]
            pltpu.sync_copy(rows_v, w_hbm.at[pl.ds(base, _RPW)])

    return gw(iid, tgt_iid_table, lin_W.reshape(_D))


def _prep_kernel(feat_ref, rpw_ref, w_ref, s_ref):
    s_ref[0:_B, :] = lax.dot_general(
        feat_ref[...], rpw_ref[...], (((1,), (1,)), ((), ())),
        preferred_element_type=jnp.float32,
        precision=lax.Precision.DEFAULT,
    )
    s_ref[_B:2 * _B, :] = w_ref[...]


def _prep(feat, rp_W, w):
    return pl.pallas_call(
        _prep_kernel,
        out_shape=jax.ShapeDtypeStruct((2 * _B, _D), jnp.float32),
    )(feat, rp_W, w)


def _score_kernel(s_ref, uid_ref, bits_ref, vt_ref):
    i = pl.program_id(0)
    av = lax.dot_general(
        s_ref[...], uid_ref[...], (((1,), (1,)), ((), ())),
        preferred_element_type=jnp.float32,
        precision=lax.Precision.DEFAULT,
    )
    a = av[0:_B, :]
    v = av[_B:2 * _B, :]
    bits = lax.bitcast_convert_type(jnp.abs(a - _TARGET), jnp.int32)
    col = i * _RB + lax.broadcasted_iota(jnp.int32, (_B, _RB), 1)
    valid = col < _N
    bits_ref[...] = jnp.where(valid, bits, _MAXBITS)
    vt_ref[...] = jnp.where(valid, v, 0.0)


def _score(s, tgt_uid_table):
    return pl.pallas_call(
        _score_kernel,
        grid=(_NBLK,),
        in_specs=[
            pl.BlockSpec((2 * _B, _D), lambda i: (0, 0)),
            pl.BlockSpec((_RB, _D), lambda i: (i, 0)),
        ],
        out_specs=[
            pl.BlockSpec((_B, _RB), lambda i: (0, i)),
            pl.BlockSpec((_B, _RB), lambda i: (0, i)),
        ],
        out_shape=[
            jax.ShapeDtypeStruct((_B, _NPAD), jnp.int32),
            jax.ShapeDtypeStruct((_B, _NPAD), jnp.float32),
        ],
    )(s, tgt_uid_table)


def _select_kernel(bits_ref, vt_ref, out_ref):
    bits = bits_ref[...]

    def body(j, p):
        test = p | jnp.left_shift(jnp.int32(1), 30 - j)
        cnt = jnp.sum((bits < test).astype(jnp.int32), axis=1, keepdims=True)
        return jnp.where(cnt < _K, test, p)

    t = lax.fori_loop(0, 31, body, jnp.zeros((bits.shape[0], 1), jnp.int32))

    v = vt_ref[...]
    lt = bits < t
    eq = bits == t
    cnt_lt = jnp.sum(lt.astype(jnp.int32), axis=1)
    cnt_eq = jnp.sum(eq.astype(jnp.int32), axis=1)
    sum_lt = jnp.sum(jnp.where(lt, v, 0.0), axis=1)
    sum_eq = jnp.sum(jnp.where(eq, v, 0.0), axis=1)
    needed = (_K - cnt_lt).astype(jnp.float32)
    res = (sum_lt + needed * sum_eq / cnt_eq.astype(jnp.float32)) / _K
    out_ref[...] = jnp.broadcast_to(res[:, None], out_ref.shape)


def _select(bits, vt):
    nprog = 4
    cb = _B // nprog
    out = pl.pallas_call(
        _select_kernel,
        grid=(nprog,),
        in_specs=[
            pl.BlockSpec((cb, _NPAD), lambda i: (i, 0)),
            pl.BlockSpec((cb, _NPAD), lambda i: (i, 0)),
        ],
        out_specs=pl.BlockSpec((cb, 128), lambda i: (i, 0)),
        out_shape=jax.ShapeDtypeStruct((_B, 128), jnp.float32),
    )(bits, vt)
    return out[:, 0]


@jax.jit
def kernel(x, tgt_uid_table, tgt_iid_table, rp_W, lin_W):
    iid = x[:, 0].astype(jnp.int32)
    feat = x[:, 1:_D + 1]
    w = _gather_w(iid, tgt_iid_table, lin_W)
    s = _prep(feat, rp_W, w)
    bits, vt = _score(s, tgt_uid_table)
    return _select(bits, vt)
